# Initial kernel scaffold; baseline (speedup 1.0000x reference)
import functools

import jax
import jax.numpy as jnp
from jax import lax
from jax.experimental import pallas as pl
from jax.experimental.pallas import tpu as pltpu
from jax.experimental.pallas import tpu_sc as plsc

N_NODES = 10000
N_PAD = 10240
D = 256
DH = 128
DT = 144
E = 160000
CHUNK = 128
NSUB = 16
NCORE = 2
EPW = 10112
E_PAD = EPW * NSUB
Z_RATIO = 0.8
GN_EPS = 1e-9

_HIGH = lax.Precision.HIGHEST


def _tableA_body(x_ref, m_ref, wt_ref, bt_ref, out_ref):
    x = x_ref[...]
    h = jnp.dot(x, wt_ref[...], precision=_HIGH) + bt_ref[...]
    h = jnp.maximum(h, 0.0)
    h1 = h[:, :D]
    h0 = h[:, D:]
    m = m_ref[...]
    c1 = (1.0 - Z_RATIO) + (2.0 * Z_RATIO - 1.0) * m
    xm = c1 * h1 + (1.0 - c1) * h0
    B = x.shape[0]
    it = lax.broadcasted_iota(jnp.int32, (B, 16), 1)
    pad = jnp.where(it == 0, 1.0, 0.0).astype(jnp.float32)
    out_ref[0, :, :DH] = xm[:, :DH]
    out_ref[0, :, DH:DT] = pad
    out_ref[1, :, :DH] = xm[:, DH:]
    out_ref[1, :, DH:DT] = pad


def _build_table(x_, maskf, Wt, bt):
    B = 1000
    grid = (N_NODES // B,)
    return pl.pallas_call(
        _tableA_body,
        grid=grid,
        in_specs=[
            pl.BlockSpec((B, D), lambda i: (i, 0)),
            pl.BlockSpec((B, 1), lambda i: (i, 0)),
            pl.BlockSpec((D, 2 * D), lambda i: (0, 0)),
            pl.BlockSpec((1, 2 * D), lambda i: (0, 0)),
        ],
        out_specs=pl.BlockSpec((2, B, DT), lambda i: (0, i, 0)),
        out_shape=jax.ShapeDtypeStruct((2, N_NODES, DT), jnp.float32),
    )(x_, maskf, Wt, bt)


def _sc_spmm(table, colp, rowp, wp):
    mesh = plsc.VectorSubcoreMesh(core_axis_name="c", subcore_axis_name="s")

    @functools.partial(
        pl.kernel,
        out_type=jax.ShapeDtypeStruct((NCORE, N_PAD, DT), jnp.float32),
        mesh=mesh,
        scratch_types=[
            pltpu.VMEM((CHUNK,), jnp.int32),
            pltpu.VMEM((CHUNK,), jnp.int32),
            pltpu.VMEM((CHUNK,), jnp.float32),
            pltpu.VMEM((CHUNK, DT), jnp.float32),
            pltpu.VMEM((CHUNK, DT), jnp.float32),
            pltpu.VMEM_SHARED((N_PAD, DT), jnp.float32),
        ],
    )
    def spmm(table_hbm, col_hbm, row_hbm, w_hbm, out_hbm,
             gidx_v, ridx_v, w_v, rows_v, zero_v, acc_sh):
        cid = lax.axis_index("c")
        sid = lax.axis_index("s")
        zero16 = jnp.zeros((16,), jnp.float32)

        @pl.loop(0, CHUNK)
        def _(r):
            for j in range(DT // 16):
                zero_v[r, pl.ds(j * 16, 16)] = zero16

        stripe = sid * 640

        @pl.loop(0, 640, step=CHUNK)
        def _(r):
            pltpu.sync_copy(zero_v, acc_sh.at[pl.ds(stripe + r, CHUNK)])

        plsc.subcore_barrier()

        base = sid * EPW
        off = cid * N_NODES

        @pl.loop(0, EPW, step=CHUNK)
        def _(e0):
            eoff = base + e0
            pltpu.sync_copy(col_hbm.at[pl.ds(eoff, CHUNK)], gidx_v)
            pltpu.sync_copy(row_hbm.at[pl.ds(eoff, CHUNK)], ridx_v)
            pltpu.sync_copy(w_hbm.at[pl.ds(eoff, CHUNK)], w_v)

            @pl.loop(0, CHUNK, step=16)
            def _(i):
                gidx_v[pl.ds(i, 16)] = gidx_v[pl.ds(i, 16)] + off

            pltpu.sync_copy(table_hbm.at[gidx_v], rows_v)

            @pl.loop(0, CHUNK)
            def _(i):
                wv = plsc.load_gather(w_v, [jnp.full((16,), i, jnp.int32)])
                for j in range(DT // 16):
                    sl = pl.ds(j * 16, 16)
                    rows_v[i, sl] = rows_v[i, sl] * wv

            pltpu.sync_copy(rows_v, acc_sh.at[ridx_v], add=True)

        plsc.subcore_barrier()

        @pl.loop(0, 640, step=CHUNK)
        def _(r):
            pltpu.sync_copy(acc_sh.at[pl.ds(stripe + r, CHUNK)],
                            out_hbm.at[cid].at[pl.ds(stripe + r, CHUNK)])

    return spmm(table, colp, rowp, wp)


def _statsB1_body(a0_ref, a1_ref, agg_ref, stats_ref, acc_ref):
    i = pl.program_id(0)
    deg = a0_ref[:, DH:DH + 1]
    deg = jnp.where(deg < 0.5, deg + 1.0, deg)
    inv = 1.0 / deg
    agg = jnp.concatenate([a0_ref[:, :DH] * inv, a1_ref[:, :DH] * inv], axis=1)
    agg_ref[...] = agg

    @pl.when(i == 0)
    def _():
        acc_ref[...] = jnp.zeros_like(acc_ref)

    acc_ref[0:1, :] += jnp.sum(agg, axis=0, keepdims=True)
    acc_ref[1:2, :] += jnp.sum(agg * agg, axis=0, keepdims=True)

    @pl.when(i == pl.num_programs(0) - 1)
    def _():
        stats_ref[...] = acc_ref[...]


def _stats(a0, a1):
    B = 1024
    grid = (N_PAD // B,)
    return pl.pallas_call(
        _statsB1_body,
        grid=grid,
        in_specs=[
            pl.BlockSpec((B, DT), lambda i: (i, 0)),
            pl.BlockSpec((B, DT), lambda i: (i, 0)),
        ],
        out_specs=[
            pl.BlockSpec((B, D), lambda i: (i, 0)),
            pl.BlockSpec((2, D), lambda i: (0, 0)),
        ],
        out_shape=[
            jax.ShapeDtypeStruct((N_PAD, D), jnp.float32),
            jax.ShapeDtypeStruct((2, D), jnp.float32),
        ],
        scratch_shapes=[pltpu.VMEM((2, D), jnp.float32)],
    )(a0, a1)


def _finalB2_body(agg_ref, x_ref, m_ref, stats_ref, wc_ref, bc_ref,
                  gs_ref, gb_ref, out_ref):
    mean = stats_ref[0:1, :] * (1.0 / N_NODES)
    ex2 = stats_ref[1:2, :] * (1.0 / N_NODES)
    var = ex2 - mean * mean
    rstd = lax.rsqrt(var + GN_EPS)
    xn = (agg_ref[...] - mean) * (rstd * gs_ref[...]) + gb_ref[...]
    z = jnp.concatenate([xn, x_ref[...]], axis=1)
    y = jnp.dot(z, wc_ref[...], precision=_HIGH) + bc_ref[...]
    y1 = y[:, :D]
    y0 = y[:, D:]
    m = m_ref[...]
    c1 = (1.0 - Z_RATIO) + (2.0 * Z_RATIO - 1.0) * m
    out_ref[...] = c1 * y1 + (1.0 - c1) * y0


def _final(agg, x_, maskf, stats, Wc, bc, gs, gb):
    B = 1000
    grid = (N_NODES // B,)
    return pl.pallas_call(
        _finalB2_body,
        grid=grid,
        in_specs=[
            pl.BlockSpec((B, D), lambda i: (i, 0)),
            pl.BlockSpec((B, D), lambda i: (i, 0)),
            pl.BlockSpec((B, 1), lambda i: (i, 0)),
            pl.BlockSpec((2, D), lambda i: (0, 0)),
            pl.BlockSpec((2 * D, 2 * D), lambda i: (0, 0)),
            pl.BlockSpec((1, 2 * D), lambda i: (0, 0)),
            pl.BlockSpec((1, D), lambda i: (0, 0)),
            pl.BlockSpec((1, D), lambda i: (0, 0)),
        ],
        out_specs=pl.BlockSpec((B, D), lambda i: (i, 0)),
        out_shape=jax.ShapeDtypeStruct((N_NODES, D), jnp.float32),
    )(agg, x_, maskf, stats, Wc, bc, gs, gb)


def kernel(x_, edge_index, edge_weight, mask, Wt0, bt0, Wt1, bt1,
           Wc0, bc0, Wc1, bc1, gn_scale, gn_bias):
    row = edge_index[0]
    col = edge_index[1]
    padn = E_PAD - E
    colp = jnp.pad(col, (0, padn))
    rowp = jnp.pad(row, (0, padn))
    wp = jnp.pad(edge_weight, (0, padn))
    maskf = mask.astype(jnp.float32)

    Wt = jnp.concatenate([Wt1, Wt0], axis=1)
    bt = jnp.concatenate([bt1, bt0]).reshape(1, 2 * D)
    Wc = jnp.concatenate([Wc1, Wc0], axis=1)
    bc = jnp.concatenate([bc1, bc0]).reshape(1, 2 * D)

    table = _build_table(x_, maskf, Wt, bt)
    acc = _sc_spmm(table.reshape(2 * N_NODES, DT), colp, rowp, wp)
    agg, stats = _stats(acc[0], acc[1])
    return _final(agg[:N_NODES], x_, maskf, stats, Wc, bc,
                  gn_scale.reshape(1, D), gn_bias.reshape(1, D))


# speedup vs baseline: 3.5260x; 3.5260x over previous
import dataclasses
import functools

import jax
import jax.numpy as jnp
from jax import lax
from jax.experimental import pallas as pl
from jax.experimental.pallas import tpu as pltpu
from jax.experimental.pallas import tpu_sc as plsc

N_NODES = 10000
N_PAD = 10240
D = 256
DH = 128
DT = 144
E = 160000
CHUNK = 128
NSUB = 16
NCORE = 2
EPW = 10112
E_PAD = EPW * NSUB
Z_RATIO = 0.8
GN_EPS = 1e-9

_HIGH = lax.Precision.HIGHEST


def _tableA_body(x_ref, m_ref, wt_ref, bt_ref, out_ref):
    x = x_ref[...]
    h = jnp.dot(x, wt_ref[...], precision=_HIGH) + bt_ref[...]
    h = jnp.maximum(h, 0.0)
    h1 = h[:, :D]
    h0 = h[:, D:]
    m = m_ref[...]
    c1 = (1.0 - Z_RATIO) + (2.0 * Z_RATIO - 1.0) * m
    xm = c1 * h1 + (1.0 - c1) * h0
    B = x.shape[0]
    it = lax.broadcasted_iota(jnp.int32, (B, 16), 1)
    pad = jnp.where(it == 0, 1.0, 0.0).astype(jnp.float32)
    out_ref[0, :, :DH] = xm[:, :DH]
    out_ref[0, :, DH:DT] = pad
    out_ref[1, :, :DH] = xm[:, DH:]
    out_ref[1, :, DH:DT] = pad


def _build_table(x_, maskf, Wt, bt):
    B = 1000
    grid = (N_NODES // B,)
    return pl.pallas_call(
        _tableA_body,
        grid=grid,
        in_specs=[
            pl.BlockSpec((B, D), lambda i: (i, 0)),
            pl.BlockSpec((B, 1), lambda i: (i, 0)),
            pl.BlockSpec((D, 2 * D), lambda i: (0, 0)),
            pl.BlockSpec((1, 2 * D), lambda i: (0, 0)),
        ],
        out_specs=pl.BlockSpec((2, B, DT), lambda i: (0, i, 0)),
        out_shape=jax.ShapeDtypeStruct((2, N_NODES, DT), jnp.float32),
    )(x_, maskf, Wt, bt)


def _sc_spmm(table, colp, rowp, wp):
    mesh = plsc.VectorSubcoreMesh(core_axis_name="c", subcore_axis_name="s")
    cp = pltpu.CompilerParams()
    if "needs_layout_passes" in pltpu.CompilerParams.__dataclass_fields__:
        cp = dataclasses.replace(cp, needs_layout_passes=False)
    if "use_tc_tiling_on_sc" in pltpu.CompilerParams.__dataclass_fields__:
        cp = dataclasses.replace(cp, use_tc_tiling_on_sc=False)

    @functools.partial(
        pl.kernel,
        out_type=jax.ShapeDtypeStruct((NCORE, N_PAD, DT), jnp.float32),
        mesh=mesh,
        compiler_params=cp,
        scratch_types=[
            pltpu.VMEM((CHUNK,), jnp.int32),
            pltpu.VMEM((CHUNK,), jnp.int32),
            pltpu.VMEM((CHUNK,), jnp.float32),
            pltpu.VMEM((CHUNK, DT), jnp.float32),
            pltpu.VMEM((CHUNK, DT), jnp.float32),
            pltpu.VMEM_SHARED((N_PAD, DT), jnp.float32),
        ],
    )
    def spmm(table_hbm, col_hbm, row_hbm, w_hbm, out_hbm,
             gidx_v, ridx_v, w_v, rows_v, zero_v, acc_sh):
        cid = lax.axis_index("c")
        sid = lax.axis_index("s")
        zero16 = jnp.zeros((16,), jnp.float32)

        @pl.loop(0, CHUNK)
        def _(r):
            for j in range(DT // 16):
                zero_v[r, pl.ds(j * 16, 16)] = zero16

        stripe = sid * 640

        @pl.loop(0, 640, step=CHUNK)
        def _(r):
            pltpu.sync_copy(zero_v, acc_sh.at[pl.ds(stripe + r, CHUNK)])

        plsc.subcore_barrier()

        base = sid * EPW
        off = cid * N_NODES

        @pl.loop(0, EPW, step=CHUNK)
        def _(e0):
            eoff = base + e0
            pltpu.sync_copy(col_hbm.at[pl.ds(eoff, CHUNK)], gidx_v)
            pltpu.sync_copy(row_hbm.at[pl.ds(eoff, CHUNK)], ridx_v)
            pltpu.sync_copy(w_hbm.at[pl.ds(eoff, CHUNK)], w_v)

            @pl.loop(0, CHUNK, step=16)
            def _(i):
                gidx_v[pl.ds(i, 16)] = gidx_v[pl.ds(i, 16)] + off

            pltpu.sync_copy(table_hbm.at[gidx_v], rows_v)

            @pl.loop(0, CHUNK)
            def _(i):
                wv = plsc.load_gather(w_v, [jnp.full((16,), i, jnp.int32)])
                for j in range(DT // 16):
                    sl = pl.ds(j * 16, 16)
                    rows_v[i, sl] = rows_v[i, sl] * wv

            pltpu.sync_copy(rows_v, acc_sh.at[ridx_v], add=True)

        plsc.subcore_barrier()

        @pl.loop(0, 640, step=CHUNK)
        def _(r):
            pltpu.sync_copy(acc_sh.at[pl.ds(stripe + r, CHUNK)],
                            out_hbm.at[cid].at[pl.ds(stripe + r, CHUNK)])

    return spmm(table, colp, rowp, wp)


def _statsB1_body(a0_ref, a1_ref, agg_ref, stats_ref, acc_ref):
    i = pl.program_id(0)
    deg = a0_ref[:, DH:DH + 1]
    deg = jnp.where(deg < 0.5, deg + 1.0, deg)
    inv = 1.0 / deg
    agg = jnp.concatenate([a0_ref[:, :DH] * inv, a1_ref[:, :DH] * inv], axis=1)
    agg_ref[...] = agg

    @pl.when(i == 0)
    def _():
        acc_ref[...] = jnp.zeros_like(acc_ref)

    acc_ref[0:1, :] += jnp.sum(agg, axis=0, keepdims=True)
    acc_ref[1:2, :] += jnp.sum(agg * agg, axis=0, keepdims=True)

    @pl.when(i == pl.num_programs(0) - 1)
    def _():
        stats_ref[...] = acc_ref[...]


def _stats(a0, a1):
    B = 1024
    grid = (N_PAD // B,)
    return pl.pallas_call(
        _statsB1_body,
        grid=grid,
        in_specs=[
            pl.BlockSpec((B, DT), lambda i: (i, 0)),
            pl.BlockSpec((B, DT), lambda i: (i, 0)),
        ],
        out_specs=[
            pl.BlockSpec((B, D), lambda i: (i, 0)),
            pl.BlockSpec((2, D), lambda i: (0, 0)),
        ],
        out_shape=[
            jax.ShapeDtypeStruct((N_PAD, D), jnp.float32),
            jax.ShapeDtypeStruct((2, D), jnp.float32),
        ],
        scratch_shapes=[pltpu.VMEM((2, D), jnp.float32)],
    )(a0, a1)


def _finalB2_body(agg_ref, x_ref, m_ref, stats_ref, wc_ref, bc_ref,
                  gs_ref, gb_ref, out_ref):
    mean = stats_ref[0:1, :] * (1.0 / N_NODES)
    ex2 = stats_ref[1:2, :] * (1.0 / N_NODES)
    var = ex2 - mean * mean
    rstd = lax.rsqrt(var + GN_EPS)
    xn = (agg_ref[...] - mean) * (rstd * gs_ref[...]) + gb_ref[...]
    z = jnp.concatenate([xn, x_ref[...]], axis=1)
    y = jnp.dot(z, wc_ref[...], precision=_HIGH) + bc_ref[...]
    y1 = y[:, :D]
    y0 = y[:, D:]
    m = m_ref[...]
    c1 = (1.0 - Z_RATIO) + (2.0 * Z_RATIO - 1.0) * m
    out_ref[...] = c1 * y1 + (1.0 - c1) * y0


def _final(agg, x_, maskf, stats, Wc, bc, gs, gb):
    B = 1000
    grid = (N_NODES // B,)
    return pl.pallas_call(
        _finalB2_body,
        grid=grid,
        in_specs=[
            pl.BlockSpec((B, D), lambda i: (i, 0)),
            pl.BlockSpec((B, D), lambda i: (i, 0)),
            pl.BlockSpec((B, 1), lambda i: (i, 0)),
            pl.BlockSpec((2, D), lambda i: (0, 0)),
            pl.BlockSpec((2 * D, 2 * D), lambda i: (0, 0)),
            pl.BlockSpec((1, 2 * D), lambda i: (0, 0)),
            pl.BlockSpec((1, D), lambda i: (0, 0)),
            pl.BlockSpec((1, D), lambda i: (0, 0)),
        ],
        out_specs=pl.BlockSpec((B, D), lambda i: (i, 0)),
        out_shape=jax.ShapeDtypeStruct((N_NODES, D), jnp.float32),
    )(agg, x_, maskf, stats, Wc, bc, gs, gb)


def kernel(x_, edge_index, edge_weight, mask, Wt0, bt0, Wt1, bt1,
           Wc0, bc0, Wc1, bc1, gn_scale, gn_bias):
    row = edge_index[0]
    col = edge_index[1]
    padn = E_PAD - E
    colp = jnp.pad(col, (0, padn))
    rowp = jnp.pad(row, (0, padn))
    wp = jnp.pad(edge_weight, (0, padn))
    maskf = mask.astype(jnp.float32)

    Wt = jnp.concatenate([Wt1, Wt0], axis=1)
    bt = jnp.concatenate([bt1, bt0]).reshape(1, 2 * D)
    Wc = jnp.concatenate([Wc1, Wc0], axis=1)
    bc = jnp.concatenate([bc1, bc0]).reshape(1, 2 * D)

    table = _build_table(x_, maskf, Wt, bt)
    acc = _sc_spmm(table.reshape(2 * N_NODES, DT), colp, rowp, wp)
    agg, stats = _stats(acc[0], acc[1])
    return _final(agg[:N_NODES], x_, maskf, stats, Wc, bc,
                  gn_scale.reshape(1, D), gn_bias.reshape(1, D))
